# baseline (device time: 48012 ns/iter reference)
import jax
import jax.numpy as jnp
from jax import lax
from jax.experimental import pallas as pl
from jax.experimental.pallas import tpu as pltpu

N_DEV = 4
E_PER = 2


def kernel(x, router_W, route_idx, expert_W, shared_W):
    m, d = x.shape
    e_per, _, h = expert_W.shape
    n_exp = router_W.shape[1]

    def body(x_ref, router_W_ref, route_idx_ref, expert_W_ref, shared_W_ref,
             out_ref, comm_ref, send_sems, recv_sems):
        my_pos = lax.axis_index("i")
        left = lax.rem(my_pos - 1 + N_DEV, N_DEV)
        right = lax.rem(my_pos + 1, N_DEV)

        comm_ref[0] = expert_W_ref[...]

        barrier_sem = pltpu.get_barrier_semaphore()
        for nbr in (left, right):
            pl.semaphore_signal(
                barrier_sem, inc=1,
                device_id=(nbr,), device_id_type=pl.DeviceIdType.MESH,
            )
        pl.semaphore_wait(barrier_sem, 2)

        def make_hop(hop):
            return pltpu.make_async_remote_copy(
                src_ref=comm_ref.at[hop],
                dst_ref=comm_ref.at[hop + 1],
                send_sem=send_sems.at[hop],
                recv_sem=recv_sems.at[hop],
                device_id=(right,),
                device_id_type=pl.DeviceIdType.MESH,
            )

        hops = [make_hop(0)]
        hops[0].start()

        x = x_ref[...]
        scores = jnp.dot(x, router_W_ref[...],
                         preferred_element_type=jnp.float32)
        s_max = jnp.max(scores, axis=-1, keepdims=True)
        p = jnp.exp(scores - s_max)
        probs = p / jnp.sum(p, axis=-1, keepdims=True)
        route = route_idx_ref[...]
        col = lax.broadcasted_iota(jnp.int32, (m, n_exp), 1)
        gate = jnp.sum(jnp.where(col == route, probs, 0.0), axis=1)

        def contrib(origin, w_slot):
            acc = jnp.zeros((m, h), jnp.float32)
            for j in range(E_PER):
                eid = origin * E_PER + j
                w = jnp.where(route[:, 0] == eid, gate, 0.0)
                acc += jnp.dot(w[:, None] * x, w_slot[j],
                               preferred_element_type=jnp.float32)
            return acc

        out = jnp.dot(x, shared_W_ref[...],
                      preferred_element_type=jnp.float32)
        out += contrib(my_pos, expert_W_ref[...])

        for hop in range(N_DEV - 1):
            hops[hop].wait_recv()
            if hop + 1 < N_DEV - 1:
                hops.append(make_hop(hop + 1))
                hops[hop + 1].start()
            origin = lax.rem(my_pos - (hop + 1) + N_DEV, N_DEV)
            out += contrib(origin, comm_ref[hop + 1])
            hops[hop].wait_send()

        out_ref[...] = out

    return pl.pallas_call(
        body,
        out_shape=jax.ShapeDtypeStruct((m, h), jnp.float32),
        in_specs=[pl.BlockSpec(memory_space=pltpu.VMEM)] * 5,
        out_specs=pl.BlockSpec(memory_space=pltpu.VMEM),
        scratch_shapes=[
            pltpu.VMEM((N_DEV, e_per, d, h), jnp.float32),
            pltpu.SemaphoreType.DMA((N_DEV - 1,)),
            pltpu.SemaphoreType.DMA((N_DEV - 1,)),
        ],
        compiler_params=pltpu.CompilerParams(collective_id=0),
    )(x, router_W, route_idx, expert_W, shared_W)


# device time: 29475 ns/iter; 1.6289x vs baseline; 1.6289x over previous
import jax
import jax.numpy as jnp
from jax import lax
from jax.experimental import pallas as pl
from jax.experimental.pallas import tpu as pltpu

N_DEV = 4
E_PER = 2


def kernel(x, router_W, route_idx, expert_W, shared_W):
    m, d = x.shape
    e_per, _, h = expert_W.shape
    n_exp = router_W.shape[1]

    def body(x_ref, router_W_ref, route_idx_ref, expert_W_ref, shared_W_ref,
             out_ref, from_l, from_r, opp, send_sems, recv_sems):
        my_pos = lax.axis_index("i")
        left = lax.rem(my_pos - 1 + N_DEV, N_DEV)
        right = lax.rem(my_pos + 1, N_DEV)

        barrier_sem = pltpu.get_barrier_semaphore()
        for nbr in (left, right):
            pl.semaphore_signal(
                barrier_sem, inc=1,
                device_id=(nbr,), device_id_type=pl.DeviceIdType.MESH,
            )
        pl.semaphore_wait(barrier_sem, 2)

        def copy(src, dst, sem, target):
            return pltpu.make_async_remote_copy(
                src_ref=src, dst_ref=dst,
                send_sem=send_sems.at[sem], recv_sem=recv_sems.at[sem],
                device_id=(target,), device_id_type=pl.DeviceIdType.MESH,
            )

        p1_r = copy(expert_W_ref, from_l, 0, right)
        p1_l = copy(expert_W_ref, from_r, 1, left)
        p1_r.start()
        p1_l.start()

        xv = x_ref[...]
        scores = jnp.dot(xv, router_W_ref[...],
                         preferred_element_type=jnp.float32)
        s_max = jnp.max(scores, axis=-1, keepdims=True)
        p = jnp.exp(scores - s_max)
        probs = p / jnp.sum(p, axis=-1, keepdims=True)
        route = route_idx_ref[...]
        col = lax.broadcasted_iota(jnp.int32, (m, n_exp), 1)
        gate = jnp.sum(jnp.where(col == route, probs, 0.0), axis=1)

        def contrib(origin, w_slot):
            acc = jnp.zeros((m, h), jnp.float32)
            for j in range(E_PER):
                eid = origin * E_PER + j
                w = jnp.where(route[:, 0] == eid, gate, 0.0)
                acc += jnp.dot(w[:, None] * xv, w_slot[j],
                               preferred_element_type=jnp.float32)
            return acc

        out = jnp.dot(xv, shared_W_ref[...],
                      preferred_element_type=jnp.float32)
        out += contrib(my_pos, expert_W_ref[...])

        p1_r.wait_recv()
        p2_r = copy(from_l.at[0], opp.at[0], 2, right)
        p2_r.start()
        out += contrib(left, from_l)

        p1_l.wait_recv()
        p2_l = copy(from_r.at[1], opp.at[1], 3, left)
        p2_l.start()
        out += contrib(right, from_r)

        opposite = lax.rem(my_pos + 2, N_DEV)
        p2_r.wait_recv()
        p2_l.wait_recv()
        out += contrib(opposite, opp)

        for rdma in (p1_r, p1_l, p2_r, p2_l):
            rdma.wait_send()
        out_ref[...] = out

    return pl.pallas_call(
        body,
        out_shape=jax.ShapeDtypeStruct((m, h), jnp.float32),
        in_specs=[pl.BlockSpec(memory_space=pltpu.VMEM)] * 5,
        out_specs=pl.BlockSpec(memory_space=pltpu.VMEM),
        scratch_shapes=[
            pltpu.VMEM((e_per, d, h), jnp.float32),
            pltpu.VMEM((e_per, d, h), jnp.float32),
            pltpu.VMEM((e_per, d, h), jnp.float32),
            pltpu.SemaphoreType.DMA((4,)),
            pltpu.SemaphoreType.DMA((4,)),
        ],
        compiler_params=pltpu.CompilerParams(collective_id=0),
    )(x, router_W, route_idx, expert_W, shared_W)


# device time: 27924 ns/iter; 1.7194x vs baseline; 1.0555x over previous
import jax
import jax.numpy as jnp
from jax import lax
from jax.experimental import pallas as pl
from jax.experimental.pallas import tpu as pltpu

N_DEV = 4
E_PER = 2


def kernel(x, router_W, route_idx, expert_W, shared_W):
    m, d = x.shape
    e_per, _, h = expert_W.shape
    n_exp = router_W.shape[1]

    def body(x_ref, router_W_ref, route_idx_ref, expert_W_ref, shared_W_ref,
             out_ref, from_l, from_r, opp, send_sems, recv_sems):
        my_pos = lax.axis_index("i")
        left = lax.rem(my_pos - 1 + N_DEV, N_DEV)
        right = lax.rem(my_pos + 1, N_DEV)

        barrier_sem = pltpu.get_barrier_semaphore()
        for nbr in (left, right):
            pl.semaphore_signal(
                barrier_sem, inc=1,
                device_id=(nbr,), device_id_type=pl.DeviceIdType.MESH,
            )
        pl.semaphore_wait(barrier_sem, 2)

        def copy(src, dst, sem, target):
            return pltpu.make_async_remote_copy(
                src_ref=src, dst_ref=dst,
                send_sem=send_sems.at[sem], recv_sem=recv_sems.at[sem],
                device_id=(target,), device_id_type=pl.DeviceIdType.MESH,
            )

        p1_r0 = copy(expert_W_ref.at[0], from_l.at[0], 0, right)
        p1_l1 = copy(expert_W_ref.at[1], from_r.at[1], 1, left)
        p1_r1 = copy(expert_W_ref.at[1], from_l.at[1], 2, right)
        p1_l0 = copy(expert_W_ref.at[0], from_r.at[0], 3, left)
        p1_r0.start()
        p1_l1.start()
        p1_r1.start()
        p1_l0.start()

        xv = x_ref[...]
        scores = jnp.dot(xv, router_W_ref[...],
                         preferred_element_type=jnp.float32)
        s_max = jnp.max(scores, axis=-1, keepdims=True)
        p = jnp.exp(scores - s_max)
        probs = p / jnp.sum(p, axis=-1, keepdims=True)
        route = route_idx_ref[...]
        col = lax.broadcasted_iota(jnp.int32, (m, n_exp), 1)
        gate = jnp.sum(jnp.where(col == route, probs, 0.0), axis=1)

        def contrib(eid, w2d):
            w = jnp.where(route[:, 0] == eid, gate, 0.0)
            return jnp.dot(w[:, None] * xv, w2d,
                           preferred_element_type=jnp.float32)

        out = jnp.dot(xv, shared_W_ref[...],
                      preferred_element_type=jnp.float32)
        out += contrib(my_pos * E_PER + 0, expert_W_ref[0])
        out += contrib(my_pos * E_PER + 1, expert_W_ref[1])

        p1_r0.wait_recv()
        p2_r = copy(from_l.at[0], opp.at[0], 4, right)
        p2_r.start()
        out += contrib(left * E_PER + 0, from_l[0])

        p1_l1.wait_recv()
        p2_l = copy(from_r.at[1], opp.at[1], 5, left)
        p2_l.start()
        out += contrib(right * E_PER + 1, from_r[1])

        p1_r1.wait_recv()
        out += contrib(left * E_PER + 1, from_l[1])
        p1_l0.wait_recv()
        out += contrib(right * E_PER + 0, from_r[0])

        opposite = lax.rem(my_pos + 2, N_DEV)
        p2_r.wait_recv()
        out += contrib(opposite * E_PER + 0, opp[0])
        p2_l.wait_recv()
        out += contrib(opposite * E_PER + 1, opp[1])

        for rdma in (p1_r0, p1_l1, p1_r1, p1_l0, p2_r, p2_l):
            rdma.wait_send()
        out_ref[...] = out

    return pl.pallas_call(
        body,
        out_shape=jax.ShapeDtypeStruct((m, h), jnp.float32),
        in_specs=[pl.BlockSpec(memory_space=pltpu.VMEM)] * 5,
        out_specs=pl.BlockSpec(memory_space=pltpu.VMEM),
        scratch_shapes=[
            pltpu.VMEM((e_per, d, h), jnp.float32),
            pltpu.VMEM((e_per, d, h), jnp.float32),
            pltpu.VMEM((e_per, d, h), jnp.float32),
            pltpu.SemaphoreType.DMA((6,)),
            pltpu.SemaphoreType.DMA((6,)),
        ],
        compiler_params=pltpu.CompilerParams(collective_id=0),
    )(x, router_W, route_idx, expert_W, shared_W)


# device time: 19537 ns/iter; 2.4575x vs baseline; 1.4293x over previous
import jax
import jax.numpy as jnp
from jax import lax
from jax.experimental import pallas as pl
from jax.experimental.pallas import tpu as pltpu

N_DEV = 4
E_PER = 2


def kernel(x, router_W, route_idx, expert_W, shared_W):
    m, d = x.shape
    e_per, _, h = expert_W.shape
    n_exp = router_W.shape[1]

    def body(x_ref, router_W_ref, route_idx_ref, expert_W_ref, shared_W_ref,
             out_ref, loc_bf, from_l, from_r, opp, send_sems, recv_sems):
        my_pos = lax.axis_index("i")
        left = lax.rem(my_pos - 1 + N_DEV, N_DEV)
        right = lax.rem(my_pos + 1, N_DEV)

        loc_bf[...] = expert_W_ref[...].astype(jnp.bfloat16)

        barrier_sem = pltpu.get_barrier_semaphore()
        for nbr in (left, right):
            pl.semaphore_signal(
                barrier_sem, inc=1,
                device_id=(nbr,), device_id_type=pl.DeviceIdType.MESH,
            )
        pl.semaphore_wait(barrier_sem, 2)

        def copy(src, dst, sem, target):
            return pltpu.make_async_remote_copy(
                src_ref=src, dst_ref=dst,
                send_sem=send_sems.at[sem], recv_sem=recv_sems.at[sem],
                device_id=(target,), device_id_type=pl.DeviceIdType.MESH,
            )

        p1_r0 = copy(loc_bf.at[0], from_l.at[0], 0, right)
        p1_l1 = copy(loc_bf.at[1], from_r.at[1], 1, left)
        p1_r1 = copy(loc_bf.at[1], from_l.at[1], 2, right)
        p1_l0 = copy(loc_bf.at[0], from_r.at[0], 3, left)
        p1_r0.start()
        p1_l1.start()
        p1_r1.start()
        p1_l0.start()

        xv = x_ref[...]
        scores = jnp.dot(xv, router_W_ref[...],
                         preferred_element_type=jnp.float32)
        s_max = jnp.max(scores, axis=-1, keepdims=True)
        p = jnp.exp(scores - s_max)
        probs = p / jnp.sum(p, axis=-1, keepdims=True)
        route = route_idx_ref[...]
        col = lax.broadcasted_iota(jnp.int32, (m, n_exp), 1)
        gate = jnp.sum(jnp.where(col == route, probs, 0.0), axis=1)

        xb = xv.astype(jnp.bfloat16)

        def contrib(eid, w2d):
            w = jnp.where(route[:, 0] == eid, gate, 0.0)
            if w2d.dtype == jnp.bfloat16:
                return jnp.dot(w.astype(jnp.bfloat16)[:, None] * xb, w2d,
                               preferred_element_type=jnp.float32)
            return jnp.dot(w[:, None] * xv, w2d,
                           preferred_element_type=jnp.float32)

        out = jnp.dot(xv, shared_W_ref[...],
                      preferred_element_type=jnp.float32)
        out += contrib(my_pos * E_PER + 0, expert_W_ref[0])
        out += contrib(my_pos * E_PER + 1, expert_W_ref[1])

        p1_r0.wait_recv()
        p2_r = copy(from_l.at[0], opp.at[0], 4, right)
        p2_r.start()
        out += contrib(left * E_PER + 0, from_l[0])

        p1_l1.wait_recv()
        p2_l = copy(from_r.at[1], opp.at[1], 5, left)
        p2_l.start()
        out += contrib(right * E_PER + 1, from_r[1])

        p1_r1.wait_recv()
        out += contrib(left * E_PER + 1, from_l[1])
        p1_l0.wait_recv()
        out += contrib(right * E_PER + 0, from_r[0])

        opposite = lax.rem(my_pos + 2, N_DEV)
        p2_r.wait_recv()
        out += contrib(opposite * E_PER + 0, opp[0])
        p2_l.wait_recv()
        out += contrib(opposite * E_PER + 1, opp[1])

        for rdma in (p1_r0, p1_l1, p1_r1, p1_l0, p2_r, p2_l):
            rdma.wait_send()
        out_ref[...] = out

    return pl.pallas_call(
        body,
        out_shape=jax.ShapeDtypeStruct((m, h), jnp.float32),
        in_specs=[pl.BlockSpec(memory_space=pltpu.VMEM)] * 5,
        out_specs=pl.BlockSpec(memory_space=pltpu.VMEM),
        scratch_shapes=[
            pltpu.VMEM((e_per, d, h), jnp.bfloat16),
            pltpu.VMEM((e_per, d, h), jnp.bfloat16),
            pltpu.VMEM((e_per, d, h), jnp.bfloat16),
            pltpu.VMEM((e_per, d, h), jnp.bfloat16),
            pltpu.SemaphoreType.DMA((6,)),
            pltpu.SemaphoreType.DMA((6,)),
        ],
        compiler_params=pltpu.CompilerParams(collective_id=0),
    )(x, router_W, route_idx, expert_W, shared_W)


# device time: 15410 ns/iter; 3.1156x vs baseline; 1.2678x over previous
import jax
import jax.numpy as jnp
from jax import lax
from jax.experimental import pallas as pl
from jax.experimental.pallas import tpu as pltpu

N_DEV = 4
E_PER = 2
COMM_DTYPE = jnp.float8_e4m3fn
W_SCALE = 32.0


def kernel(x, router_W, route_idx, expert_W, shared_W):
    m, d = x.shape
    e_per, _, h = expert_W.shape
    n_exp = router_W.shape[1]

    def body(x_ref, router_W_ref, route_idx_ref, expert_W_ref, shared_W_ref,
             out_ref, loc_bf, from_l, from_r, opp, send_sems, recv_sems):
        my_pos = lax.axis_index("i")
        left = lax.rem(my_pos - 1 + N_DEV, N_DEV)
        right = lax.rem(my_pos + 1, N_DEV)

        loc_bf[...] = (expert_W_ref[...] * W_SCALE).astype(COMM_DTYPE)

        barrier_sem = pltpu.get_barrier_semaphore()
        for nbr in (left, right):
            pl.semaphore_signal(
                barrier_sem, inc=1,
                device_id=(nbr,), device_id_type=pl.DeviceIdType.MESH,
            )
        pl.semaphore_wait(barrier_sem, 2)

        def copy(src, dst, sem, target):
            return pltpu.make_async_remote_copy(
                src_ref=src, dst_ref=dst,
                send_sem=send_sems.at[sem], recv_sem=recv_sems.at[sem],
                device_id=(target,), device_id_type=pl.DeviceIdType.MESH,
            )

        p1_r0 = copy(loc_bf.at[0], from_l.at[0], 0, right)
        p1_l1 = copy(loc_bf.at[1], from_r.at[1], 1, left)
        p1_r1 = copy(loc_bf.at[1], from_l.at[1], 2, right)
        p1_l0 = copy(loc_bf.at[0], from_r.at[0], 3, left)
        p1_r0.start()
        p1_l1.start()
        p1_r1.start()
        p1_l0.start()

        xv = x_ref[...]
        scores = jnp.dot(xv, router_W_ref[...],
                         preferred_element_type=jnp.float32)
        s_max = jnp.max(scores, axis=-1, keepdims=True)
        p = jnp.exp(scores - s_max)
        probs = p / jnp.sum(p, axis=-1, keepdims=True)
        route = route_idx_ref[...]
        col = lax.broadcasted_iota(jnp.int32, (m, n_exp), 1)
        gate = jnp.sum(jnp.where(col == route, probs, 0.0), axis=1)

        xb = xv.astype(jnp.bfloat16)

        def contrib(eid, w2d):
            w = jnp.where(route[:, 0] == eid, gate, 0.0)
            if w2d.dtype == COMM_DTYPE:
                wb = (w * (1.0 / W_SCALE)).astype(jnp.bfloat16)
                return jnp.dot(wb[:, None] * xb, w2d.astype(jnp.bfloat16),
                               preferred_element_type=jnp.float32)
            return jnp.dot(w[:, None] * xv, w2d,
                           preferred_element_type=jnp.float32)

        out = jnp.dot(xv, shared_W_ref[...],
                      preferred_element_type=jnp.float32)
        out += contrib(my_pos * E_PER + 0, expert_W_ref[0])
        out += contrib(my_pos * E_PER + 1, expert_W_ref[1])

        p1_r0.wait_recv()
        p2_r = copy(from_l.at[0], opp.at[0], 4, right)
        p2_r.start()
        out += contrib(left * E_PER + 0, from_l[0])

        p1_l1.wait_recv()
        p2_l = copy(from_r.at[1], opp.at[1], 5, left)
        p2_l.start()
        out += contrib(right * E_PER + 1, from_r[1])

        p1_r1.wait_recv()
        out += contrib(left * E_PER + 1, from_l[1])
        p1_l0.wait_recv()
        out += contrib(right * E_PER + 0, from_r[0])

        opposite = lax.rem(my_pos + 2, N_DEV)
        p2_r.wait_recv()
        out += contrib(opposite * E_PER + 0, opp[0])
        p2_l.wait_recv()
        out += contrib(opposite * E_PER + 1, opp[1])

        for rdma in (p1_r0, p1_l1, p1_r1, p1_l0, p2_r, p2_l):
            rdma.wait_send()
        out_ref[...] = out

    return pl.pallas_call(
        body,
        out_shape=jax.ShapeDtypeStruct((m, h), jnp.float32),
        in_specs=[pl.BlockSpec(memory_space=pltpu.VMEM)] * 5,
        out_specs=pl.BlockSpec(memory_space=pltpu.VMEM),
        scratch_shapes=[
            pltpu.VMEM((e_per, d, h), COMM_DTYPE),
            pltpu.VMEM((e_per, d, h), COMM_DTYPE),
            pltpu.VMEM((e_per, d, h), COMM_DTYPE),
            pltpu.VMEM((e_per, d, h), COMM_DTYPE),
            pltpu.SemaphoreType.DMA((6,)),
            pltpu.SemaphoreType.DMA((6,)),
        ],
        compiler_params=pltpu.CompilerParams(collective_id=0),
    )(x, router_W, route_idx, expert_W, shared_W)


# device time: 15368 ns/iter; 3.1242x vs baseline; 1.0027x over previous
import jax
import jax.numpy as jnp
from jax import lax
from jax.experimental import pallas as pl
from jax.experimental.pallas import tpu as pltpu

N_DEV = 4
E_PER = 2
COMM_DTYPE = jnp.float8_e4m3fn
W_SCALE = 32.0


def kernel(x, router_W, route_idx, expert_W, shared_W):
    m, d = x.shape
    e_per, _, h = expert_W.shape
    n_exp = router_W.shape[1]

    def body(x_ref, router_W_ref, route_idx_ref, expert_W_ref, shared_W_ref,
             out_ref, loc_bf, from_l, from_r, opp, send_sems, recv_sems):
        my_pos = lax.axis_index("i")
        left = lax.rem(my_pos - 1 + N_DEV, N_DEV)
        right = lax.rem(my_pos + 1, N_DEV)

        loc_bf[...] = (expert_W_ref[...] * W_SCALE).astype(COMM_DTYPE)

        barrier_sem = pltpu.get_barrier_semaphore()
        for nbr in (left, right):
            pl.semaphore_signal(
                barrier_sem, inc=1,
                device_id=(nbr,), device_id_type=pl.DeviceIdType.MESH,
            )
        pl.semaphore_wait(barrier_sem, 2)

        def copy(src, dst, sem, target):
            return pltpu.make_async_remote_copy(
                src_ref=src, dst_ref=dst,
                send_sem=send_sems.at[sem], recv_sem=recv_sems.at[sem],
                device_id=(target,), device_id_type=pl.DeviceIdType.MESH,
            )

        p1_r0 = copy(loc_bf.at[0], from_l.at[0], 0, right)
        p1_l1 = copy(loc_bf.at[1], from_r.at[1], 1, left)
        p1_r1 = copy(loc_bf.at[1], from_l.at[1], 2, right)
        p1_l0 = copy(loc_bf.at[0], from_r.at[0], 3, left)
        p1_r0.start()
        p1_l1.start()
        p1_r1.start()
        p1_l0.start()

        xv = x_ref[...]
        scores = jnp.dot(xv, router_W_ref[...],
                         preferred_element_type=jnp.float32)
        s_max = jnp.max(scores, axis=-1, keepdims=True)
        p = jnp.exp(scores - s_max)
        probs = p / jnp.sum(p, axis=-1, keepdims=True)
        route = route_idx_ref[...]
        col = lax.broadcasted_iota(jnp.int32, (m, n_exp), 1)
        gate = jnp.sum(jnp.where(col == route, probs, 0.0), axis=1)

        xb = xv.astype(jnp.bfloat16)

        def contrib(eid, w2d):
            w = jnp.where(route[:, 0] == eid, gate, 0.0)
            if w2d.dtype == COMM_DTYPE:
                wb = (w * (1.0 / W_SCALE)).astype(jnp.bfloat16)
                return jnp.dot(wb[:, None] * xb, w2d.astype(jnp.bfloat16),
                               preferred_element_type=jnp.float32)
            return jnp.dot(w[:, None] * xv, w2d,
                           preferred_element_type=jnp.float32)

        out = jnp.dot(xv, shared_W_ref[...],
                      preferred_element_type=jnp.float32)
        out += contrib(my_pos * E_PER + 0, expert_W_ref[0])
        out += contrib(my_pos * E_PER + 1, expert_W_ref[1])

        p1_r0.wait_recv()
        p2_r = copy(from_l.at[0], opp.at[0], 4, right)
        p2_r.start()
        p1_l1.wait_recv()
        p2_l = copy(from_r.at[1], opp.at[1], 5, left)
        p2_l.start()

        out += contrib(left * E_PER + 0, from_l[0])
        out += contrib(right * E_PER + 1, from_r[1])

        p1_r1.wait_recv()
        out += contrib(left * E_PER + 1, from_l[1])
        p1_l0.wait_recv()
        out += contrib(right * E_PER + 0, from_r[0])

        opposite = lax.rem(my_pos + 2, N_DEV)
        p2_r.wait_recv()
        out += contrib(opposite * E_PER + 0, opp[0])
        p2_l.wait_recv()
        out += contrib(opposite * E_PER + 1, opp[1])

        for rdma in (p1_r0, p1_l1, p1_r1, p1_l0, p2_r, p2_l):
            rdma.wait_send()
        out_ref[...] = out

    return pl.pallas_call(
        body,
        out_shape=jax.ShapeDtypeStruct((m, h), jnp.float32),
        in_specs=[pl.BlockSpec(memory_space=pltpu.VMEM)] * 5,
        out_specs=pl.BlockSpec(memory_space=pltpu.VMEM),
        scratch_shapes=[
            pltpu.VMEM((e_per, d, h), COMM_DTYPE),
            pltpu.VMEM((e_per, d, h), COMM_DTYPE),
            pltpu.VMEM((e_per, d, h), COMM_DTYPE),
            pltpu.VMEM((e_per, d, h), COMM_DTYPE),
            pltpu.SemaphoreType.DMA((6,)),
            pltpu.SemaphoreType.DMA((6,)),
        ],
        compiler_params=pltpu.CompilerParams(collective_id=0),
    )(x, router_W, route_idx, expert_W, shared_W)


# device time: 15339 ns/iter; 3.1301x vs baseline; 1.0019x over previous
import jax
import jax.numpy as jnp
from jax import lax
from jax.experimental import pallas as pl
from jax.experimental.pallas import tpu as pltpu

N_DEV = 4
E_PER = 2
COMM_DTYPE = jnp.float8_e4m3fn
W_SCALE = 32.0


def kernel(x, router_W, route_idx, expert_W, shared_W):
    m, d = x.shape
    e_per, _, h = expert_W.shape
    n_exp = router_W.shape[1]

    def body(x_hbm, router_W_ref, route_idx_ref, ew_hbm, sw_hbm,
             out_hbm, x_s, ew_s, sw_s, out_s, loc_bf, from_l, from_r, opp,
             dma_sems, send_sems, recv_sems):
        my_pos = lax.axis_index("i")
        left = lax.rem(my_pos - 1 + N_DEV, N_DEV)
        right = lax.rem(my_pos + 1, N_DEV)

        barrier_sem = pltpu.get_barrier_semaphore()
        for nbr in (left, right):
            pl.semaphore_signal(
                barrier_sem, inc=1,
                device_id=(nbr,), device_id_type=pl.DeviceIdType.MESH,
            )

        ew_dma = pltpu.make_async_copy(ew_hbm, ew_s, dma_sems.at[0])
        x_dma = pltpu.make_async_copy(x_hbm, x_s, dma_sems.at[1])
        sw_dma = pltpu.make_async_copy(sw_hbm, sw_s, dma_sems.at[2])
        ew_dma.start()
        x_dma.start()
        sw_dma.start()

        ew_dma.wait()
        loc_bf[...] = (ew_s[...] * W_SCALE).astype(COMM_DTYPE)

        pl.semaphore_wait(barrier_sem, 2)

        def copy(src, dst, sem, target):
            return pltpu.make_async_remote_copy(
                src_ref=src, dst_ref=dst,
                send_sem=send_sems.at[sem], recv_sem=recv_sems.at[sem],
                device_id=(target,), device_id_type=pl.DeviceIdType.MESH,
            )

        p1_r0 = copy(loc_bf.at[0], from_l.at[0], 0, right)
        p1_l1 = copy(loc_bf.at[1], from_r.at[1], 1, left)
        p1_r1 = copy(loc_bf.at[1], from_l.at[1], 2, right)
        p1_l0 = copy(loc_bf.at[0], from_r.at[0], 3, left)
        p1_r0.start()
        p1_l1.start()
        p1_r1.start()
        p1_l0.start()

        x_dma.wait()
        xv = x_s[...]
        scores = jnp.dot(xv, router_W_ref[...],
                         preferred_element_type=jnp.float32)
        s_max = jnp.max(scores, axis=-1, keepdims=True)
        p = jnp.exp(scores - s_max)
        probs = p / jnp.sum(p, axis=-1, keepdims=True)
        route = route_idx_ref[...]
        col = lax.broadcasted_iota(jnp.int32, (m, n_exp), 1)
        gate = jnp.sum(jnp.where(col == route, probs, 0.0), axis=1)

        xb = xv.astype(jnp.bfloat16)

        def contrib(eid, w2d):
            w = jnp.where(route[:, 0] == eid, gate, 0.0)
            if w2d.dtype == COMM_DTYPE:
                wb = (w * (1.0 / W_SCALE)).astype(jnp.bfloat16)
                return jnp.dot(wb[:, None] * xb, w2d.astype(jnp.bfloat16),
                               preferred_element_type=jnp.float32)
            return jnp.dot(w[:, None] * xv, w2d,
                           preferred_element_type=jnp.float32)

        sw_dma.wait()
        out = jnp.dot(xv, sw_s[...], preferred_element_type=jnp.float32)
        out += contrib(my_pos * E_PER + 0, ew_s[0])
        out += contrib(my_pos * E_PER + 1, ew_s[1])

        p1_r0.wait_recv()
        p2_r = copy(from_l.at[0], opp.at[0], 4, right)
        p2_r.start()
        p1_l1.wait_recv()
        p2_l = copy(from_r.at[1], opp.at[1], 5, left)
        p2_l.start()

        out += contrib(left * E_PER + 0, from_l[0])
        out += contrib(right * E_PER + 1, from_r[1])

        p1_r1.wait_recv()
        out += contrib(left * E_PER + 1, from_l[1])
        p1_l0.wait_recv()
        out += contrib(right * E_PER + 0, from_r[0])

        opposite = lax.rem(my_pos + 2, N_DEV)
        p2_r.wait_recv()
        out += contrib(opposite * E_PER + 0, opp[0])
        p2_l.wait_recv()
        out += contrib(opposite * E_PER + 1, opp[1])

        out_s[...] = out
        out_dma = pltpu.make_async_copy(out_s, out_hbm, dma_sems.at[3])
        out_dma.start()
        for rdma in (p1_r0, p1_l1, p1_r1, p1_l0, p2_r, p2_l):
            rdma.wait_send()
        out_dma.wait()

    return pl.pallas_call(
        body,
        out_shape=jax.ShapeDtypeStruct((m, h), jnp.float32),
        in_specs=[
            pl.BlockSpec(memory_space=pl.ANY),
            pl.BlockSpec(memory_space=pltpu.VMEM),
            pl.BlockSpec(memory_space=pltpu.VMEM),
            pl.BlockSpec(memory_space=pl.ANY),
            pl.BlockSpec(memory_space=pl.ANY),
        ],
        out_specs=pl.BlockSpec(memory_space=pl.ANY),
        scratch_shapes=[
            pltpu.VMEM((m, d), jnp.float32),
            pltpu.VMEM((e_per, d, h), jnp.float32),
            pltpu.VMEM((d, h), jnp.float32),
            pltpu.VMEM((m, h), jnp.float32),
            pltpu.VMEM((e_per, d, h), COMM_DTYPE),
            pltpu.VMEM((e_per, d, h), COMM_DTYPE),
            pltpu.VMEM((e_per, d, h), COMM_DTYPE),
            pltpu.VMEM((e_per, d, h), COMM_DTYPE),
            pltpu.SemaphoreType.DMA((4,)),
            pltpu.SemaphoreType.DMA((6,)),
            pltpu.SemaphoreType.DMA((6,)),
        ],
        compiler_params=pltpu.CompilerParams(collective_id=0),
    )(x, router_W, route_idx, expert_W, shared_W)
